# baseline (speedup 1.0000x reference)
import functools

import jax
import jax.numpy as jnp
from jax import lax
from jax.experimental import pallas as pl
from jax.experimental.pallas import tpu as pltpu
from jax.experimental.pallas import tpu_sc as plsc

_NC = 2
_NS = 16
_NW = _NC * _NS
_L = 16


def _round_up(a, b):
    return (a + b - 1) // b * b


def _proj_body(x_ref, w1_ref, b1_ref, w2_ref, s_ref):
    xb = x_ref[...]
    h = jnp.maximum(
        lax.dot_general(xb, w1_ref[...], (((1,), (1,)), ((), ())),
                        preferred_element_type=jnp.float32)
        + b1_ref[...][None, :],
        0.0,
    )
    s_ref[...] = lax.dot_general(h, w2_ref[...], (((1,), (1,)), ((), ())),
                                 preferred_element_type=jnp.float32)


def _node_scalar(x, W1, b1, W2, block_rows):
    n, d_in = x.shape
    d_hid = W1.shape[0]
    grid = n // block_rows
    return pl.pallas_call(
        _proj_body,
        grid=(grid,),
        in_specs=[
            pl.BlockSpec((block_rows, d_in), lambda i: (i, 0)),
            pl.BlockSpec((d_hid, d_in), lambda i: (0, 0)),
            pl.BlockSpec((d_hid,), lambda i: (0,)),
            pl.BlockSpec((1, d_hid), lambda i: (0, 0)),
        ],
        out_specs=pl.BlockSpec((block_rows, 1), lambda i: (i, 0)),
        out_shape=jax.ShapeDtypeStruct((n, 1), jnp.float32),
    )(x, W1, b1, W2)


_UNROLL = 4


def _make_sc_scatter(n_nodes, n_pad, ept):
    mesh = plsc.VectorSubcoreMesh(
        core_axis_name="c", subcore_axis_name="s",
        num_cores=_NC, num_subcores=_NS)

    ept_pad = _round_up(ept, _L)
    n_full = ept // _L
    rem = ept - n_full * _L
    rows = n_pad // _NS

    @functools.partial(
        pl.kernel,
        out_type=[
            jax.ShapeDtypeStruct((_NC, n_pad), jnp.float32),
            jax.ShapeDtypeStruct((_NC, n_pad), jnp.float32),
        ],
        mesh=mesh,
        scratch_types=[
            pltpu.VMEM((n_nodes,), jnp.float32),
            pltpu.VMEM((n_pad,), jnp.float32),
            pltpu.VMEM((n_pad,), jnp.float32),
            pltpu.VMEM((ept_pad,), jnp.int32),
            pltpu.VMEM((ept_pad,), jnp.int32),
            pltpu.VMEM((_NS, rows), jnp.float32),
            pltpu.VMEM((rows,), jnp.float32),
            pltpu.VMEM_SHARED((_NS, n_pad), jnp.float32),
            pltpu.VMEM_SHARED((_NS, n_pad), jnp.float32),
            pltpu.SemaphoreType.DMA,
        ],
        compiler_params=pltpu.CompilerParams(
            use_tc_tiling_on_sc=False, needs_layout_passes=False),
    )
    def sc_scatter(s_hbm, src_hbm, dst_hbm, sums_hbm, cnts_hbm,
                   s_v, acc_v, cnt_v, src_v, dst_v,
                   red_v, out_v, sh_acc, sh_cnt, sem):
        cid = lax.axis_index("c")
        sid = lax.axis_index("s")
        wid = sid * _NC + cid
        base = wid * ept

        cp_s = pltpu.async_copy(s_hbm, s_v, sem)
        cp_src = pltpu.async_copy(src_hbm.at[pl.ds(base, ept)],
                                  src_v.at[pl.ds(0, ept)], sem)
        cp_dst = pltpu.async_copy(dst_hbm.at[pl.ds(base, ept)],
                                  dst_v.at[pl.ds(0, ept)], sem)

        zeros = jnp.zeros((_L,), jnp.float32)

        def zero_body(i, carry):
            for u in range(_UNROLL):
                acc_v[pl.ds((i * _UNROLL + u) * _L, _L)] = zeros
                cnt_v[pl.ds((i * _UNROLL + u) * _L, _L)] = zeros
            return carry

        lax.fori_loop(0, n_pad // (_L * _UNROLL), zero_body, 0)

        cp_s.wait()
        cp_src.wait()
        cp_dst.wait()

        ones = jnp.ones((_L,), jnp.float32)

        def edge_group(g):
            sidx = src_v[pl.ds(g * _L, _L)]
            didx = dst_v[pl.ds(g * _L, _L)]
            vals = plsc.load_gather(s_v, [sidx])
            plsc.addupdate_scatter(acc_v, [didx], vals)
            plsc.addupdate_scatter(cnt_v, [didx], ones)

        def edge_body(i, carry):
            for u in range(_UNROLL):
                edge_group(i * _UNROLL + u)
            return carry

        lax.fori_loop(0, n_full // _UNROLL, edge_body, 0)
        for g in range(n_full // _UNROLL * _UNROLL, n_full):
            edge_group(g)

        if rem:
            lane = lax.iota(jnp.int32, _L)
            valid = lane < rem
            sidx = jnp.where(valid, src_v[pl.ds(n_full * _L, _L)], 0)
            didx = jnp.where(valid, dst_v[pl.ds(n_full * _L, _L)], n_nodes)
            vals = plsc.load_gather(s_v, [sidx])
            plsc.addupdate_scatter(acc_v, [didx], vals)
            plsc.addupdate_scatter(
                cnt_v, [didx], jnp.where(valid, ones, jnp.zeros((_L,))))

        pltpu.sync_copy(acc_v, sh_acc.at[sid])
        pltpu.sync_copy(cnt_v, sh_cnt.at[sid])
        plsc.subcore_barrier()

        def reduce_one(sh, out_hbm):
            pltpu.sync_copy(sh.at[:, pl.ds(sid * rows, rows)], red_v)

            def red_body(j, carry):
                v = red_v[0, pl.ds(j * _L, _L)]
                for r in range(1, _NS):
                    v = v + red_v[r, pl.ds(j * _L, _L)]
                out_v[pl.ds(j * _L, _L)] = v
                return carry

            lax.fori_loop(0, rows // _L, red_body, 0)
            pltpu.sync_copy(out_v, out_hbm.at[cid, pl.ds(sid * rows, rows)])

        reduce_one(sh_acc, sums_hbm)
        reduce_one(sh_cnt, cnts_hbm)

    return sc_scatter


def _finalize_body(sums_ref, cnts_ref, b2_ref, out_ref):
    tot = jnp.sum(sums_ref[...], axis=0, keepdims=True)
    cnt = jnp.sum(cnts_ref[...], axis=0, keepdims=True)
    out_ref[...] = tot / jnp.maximum(cnt, 1.0) + b2_ref[0, 0]


def _finalize(sums, cnts, b2, n_pad):
    return pl.pallas_call(
        _finalize_body,
        in_specs=[
            pl.BlockSpec(memory_space=pltpu.VMEM),
            pl.BlockSpec(memory_space=pltpu.VMEM),
            pl.BlockSpec(memory_space=pltpu.SMEM),
        ],
        out_specs=pl.BlockSpec(memory_space=pltpu.VMEM),
        out_shape=jax.ShapeDtypeStruct((1, n_pad), jnp.float32),
    )(sums, cnts, b2.reshape(1, 1))


@jax.jit
def kernel(x, edge_index, W1, b1, W2, b2):
    n, _ = x.shape
    e = edge_index.shape[1]

    ept = -(-e // _NW)
    n_pad = _round_up(n + 1, 1024)

    s = _node_scalar(x, W1, b1, W2, block_rows=2000)
    s_flat = s.reshape(n)

    src = edge_index[0].astype(jnp.int32)
    dst = edge_index[1].astype(jnp.int32)

    sums, cnts = _make_sc_scatter(n, n_pad, ept)(s_flat, src, dst)

    out_pad = _finalize(sums, cnts, b2, n_pad)
    return out_pad.reshape(n_pad, 1)[:n]


# speedup vs baseline: 1.0505x; 1.0505x over previous
import functools

import jax
import jax.numpy as jnp
from jax import lax
from jax.experimental import pallas as pl
from jax.experimental.pallas import tpu as pltpu
from jax.experimental.pallas import tpu_sc as plsc

_NC = 2
_NS = 16
_NW = _NC * _NS
_L = 16


def _round_up(a, b):
    return (a + b - 1) // b * b


def _proj_body(x_ref, w1_ref, b1_ref, w2_ref, s_ref):
    xb = x_ref[...]
    h = jnp.maximum(
        lax.dot_general(xb, w1_ref[...], (((1,), (1,)), ((), ())),
                        preferred_element_type=jnp.float32)
        + b1_ref[...][None, :],
        0.0,
    )
    s_ref[...] = lax.dot_general(h, w2_ref[...], (((1,), (1,)), ((), ())),
                                 preferred_element_type=jnp.float32)


def _node_scalar(x, W1, b1, W2, block_rows):
    n, d_in = x.shape
    d_hid = W1.shape[0]
    grid = n // block_rows
    return pl.pallas_call(
        _proj_body,
        grid=(grid,),
        in_specs=[
            pl.BlockSpec((block_rows, d_in), lambda i: (i, 0)),
            pl.BlockSpec((d_hid, d_in), lambda i: (0, 0)),
            pl.BlockSpec((d_hid,), lambda i: (0,)),
            pl.BlockSpec((1, d_hid), lambda i: (0, 0)),
        ],
        out_specs=pl.BlockSpec((block_rows, 1), lambda i: (i, 0)),
        out_shape=jax.ShapeDtypeStruct((n, 1), jnp.float32),
    )(x, W1, b1, W2)


_UNROLL = 4


def _make_sc_scatter(n_nodes, n_pad, ept):
    mesh = plsc.VectorSubcoreMesh(
        core_axis_name="c", subcore_axis_name="s",
        num_cores=1, num_subcores=_NS)

    ept_pad = _round_up(ept, _L)
    n_full = ept // _L
    rem = ept - n_full * _L
    rows = n_pad // _NS

    @functools.partial(
        pl.kernel,
        out_type=jax.ShapeDtypeStruct((n_pad,), jnp.float32),
        mesh=mesh,
        scratch_types=[
            pltpu.VMEM((n_nodes,), jnp.float32),
            pltpu.VMEM((n_pad,), jnp.float32),
            pltpu.VMEM((n_pad,), jnp.float32),
            pltpu.VMEM((ept_pad,), jnp.int32),
            pltpu.VMEM((ept_pad,), jnp.int32),
            pltpu.VMEM((_NS, rows), jnp.float32),
            pltpu.VMEM((rows,), jnp.float32),
            pltpu.VMEM((_L,), jnp.float32),
            pltpu.VMEM_SHARED((_NS, n_pad), jnp.float32),
            pltpu.VMEM_SHARED((_NS, n_pad), jnp.float32),
            pltpu.SemaphoreType.DMA,
        ],
        compiler_params=pltpu.CompilerParams(
            use_tc_tiling_on_sc=False, needs_layout_passes=False),
    )
    def sc_scatter(s_hbm, src_hbm, dst_hbm, b2_hbm, out_hbm,
                   s_v, acc_v, cnt_v, src_v, dst_v,
                   red_v, out_v, b2_v, sh_acc, sh_cnt, sem):
        sid = lax.axis_index("s")
        base = sid * ept

        cp_s = pltpu.async_copy(s_hbm, s_v, sem)
        cp_src = pltpu.async_copy(src_hbm.at[pl.ds(base, ept)],
                                  src_v.at[pl.ds(0, ept)], sem)
        cp_dst = pltpu.async_copy(dst_hbm.at[pl.ds(base, ept)],
                                  dst_v.at[pl.ds(0, ept)], sem)
        cp_b2 = pltpu.async_copy(b2_hbm, b2_v, sem)

        zeros = jnp.zeros((_L,), jnp.float32)

        def zero_body(i, carry):
            for u in range(_UNROLL):
                acc_v[pl.ds((i * _UNROLL + u) * _L, _L)] = zeros
                cnt_v[pl.ds((i * _UNROLL + u) * _L, _L)] = zeros
            return carry

        lax.fori_loop(0, n_pad // (_L * _UNROLL), zero_body, 0)

        cp_s.wait()
        cp_src.wait()
        cp_dst.wait()
        cp_b2.wait()

        ones = jnp.ones((_L,), jnp.float32)

        def edge_group(g):
            sidx = src_v[pl.ds(g * _L, _L)]
            didx = dst_v[pl.ds(g * _L, _L)]
            vals = plsc.load_gather(s_v, [sidx])
            plsc.addupdate_scatter(acc_v, [didx], vals)
            plsc.addupdate_scatter(cnt_v, [didx], ones)

        def edge_body(i, carry):
            for u in range(_UNROLL):
                edge_group(i * _UNROLL + u)
            return carry

        lax.fori_loop(0, n_full // _UNROLL, edge_body, 0)
        for g in range(n_full // _UNROLL * _UNROLL, n_full):
            edge_group(g)

        if rem:
            lane = lax.iota(jnp.int32, _L)
            valid = lane < rem
            sidx = jnp.where(valid, src_v[pl.ds(n_full * _L, _L)], 0)
            didx = jnp.where(valid, dst_v[pl.ds(n_full * _L, _L)], n_nodes)
            vals = plsc.load_gather(s_v, [sidx])
            plsc.addupdate_scatter(acc_v, [didx], vals)
            plsc.addupdate_scatter(
                cnt_v, [didx], jnp.where(valid, ones, jnp.zeros((_L,))))

        pltpu.sync_copy(acc_v, sh_acc.at[sid])
        pltpu.sync_copy(cnt_v, sh_cnt.at[sid])
        plsc.subcore_barrier()

        pltpu.sync_copy(sh_acc.at[:, pl.ds(sid * rows, rows)], red_v)

        def red_sum_body(j, carry):
            v = red_v[0, pl.ds(j * _L, _L)]
            for r in range(1, _NS):
                v = v + red_v[r, pl.ds(j * _L, _L)]
            out_v[pl.ds(j * _L, _L)] = v
            return carry

        lax.fori_loop(0, rows // _L, red_sum_body, 0)

        pltpu.sync_copy(sh_cnt.at[:, pl.ds(sid * rows, rows)], red_v)
        b2v = b2_v[...]

        def red_cnt_body(j, carry):
            c = red_v[0, pl.ds(j * _L, _L)]
            for r in range(1, _NS):
                c = c + red_v[r, pl.ds(j * _L, _L)]
            sl = pl.ds(j * _L, _L)
            out_v[sl] = out_v[sl] / jnp.maximum(c, 1.0) + b2v
            return carry

        lax.fori_loop(0, rows // _L, red_cnt_body, 0)
        pltpu.sync_copy(out_v, out_hbm.at[pl.ds(sid * rows, rows)])

    return sc_scatter


@jax.jit
def kernel(x, edge_index, W1, b1, W2, b2):
    n, _ = x.shape
    e = edge_index.shape[1]

    ept = -(-e // _NS)
    n_pad = _round_up(n + 1, 1024)

    s = _node_scalar(x, W1, b1, W2, block_rows=2000)
    s_flat = s.reshape(n)

    src = edge_index[0].astype(jnp.int32)
    dst = edge_index[1].astype(jnp.int32)
    b2v = jnp.broadcast_to(b2.astype(jnp.float32), (_L,))

    out_vec = _make_sc_scatter(n, n_pad, ept)(s_flat, src, dst, b2v)
    return out_vec[:n, None]


# speedup vs baseline: 1.0537x; 1.0030x over previous
import functools

import jax
import jax.numpy as jnp
from jax import lax
from jax.experimental import pallas as pl
from jax.experimental.pallas import tpu as pltpu
from jax.experimental.pallas import tpu_sc as plsc

_NC = 2
_NS = 16
_NW = _NC * _NS
_L = 16


def _round_up(a, b):
    return (a + b - 1) // b * b


def _proj_body(x_ref, w1_ref, b1_ref, w2_ref, s_ref):
    xb = x_ref[...]
    h = jnp.maximum(
        lax.dot_general(xb, w1_ref[...], (((1,), (1,)), ((), ())),
                        preferred_element_type=jnp.float32)
        + b1_ref[...][None, :],
        0.0,
    )
    s_ref[...] = lax.dot_general(h, w2_ref[...], (((1,), (1,)), ((), ())),
                                 preferred_element_type=jnp.float32)


def _node_scalar(x, W1, b1, W2, block_rows):
    n, d_in = x.shape
    d_hid = W1.shape[0]
    grid = n // block_rows
    return pl.pallas_call(
        _proj_body,
        grid=(grid,),
        in_specs=[
            pl.BlockSpec((block_rows, d_in), lambda i: (i, 0)),
            pl.BlockSpec((d_hid, d_in), lambda i: (0, 0)),
            pl.BlockSpec((d_hid,), lambda i: (0,)),
            pl.BlockSpec((1, d_hid), lambda i: (0, 0)),
        ],
        out_specs=pl.BlockSpec((block_rows, 1), lambda i: (i, 0)),
        out_shape=jax.ShapeDtypeStruct((n, 1), jnp.float32),
    )(x, W1, b1, W2)


_UNROLL = 8


def _make_sc_scatter(n_nodes, n_pad, ept):
    mesh = plsc.VectorSubcoreMesh(
        core_axis_name="c", subcore_axis_name="s",
        num_cores=1, num_subcores=_NS)

    ept_pad = _round_up(ept, _L)
    n_full = ept // _L
    rem = ept - n_full * _L
    rows = n_pad // _NS

    tail_rows = n_nodes - (_NS - 1) * rows

    @functools.partial(
        pl.kernel,
        out_type=jax.ShapeDtypeStruct((n_nodes,), jnp.float32),
        mesh=mesh,
        scratch_types=[
            pltpu.VMEM((n_nodes,), jnp.float32),
            pltpu.VMEM((n_pad,), jnp.float32),
            pltpu.VMEM((n_pad,), jnp.float32),
            pltpu.VMEM((ept_pad,), jnp.int32),
            pltpu.VMEM((ept_pad,), jnp.int32),
            pltpu.VMEM((_NS, rows), jnp.float32),
            pltpu.VMEM((rows,), jnp.float32),
            pltpu.VMEM((_L,), jnp.float32),
            pltpu.VMEM_SHARED((_NS, n_pad), jnp.float32),
            pltpu.VMEM_SHARED((_NS, n_pad), jnp.float32),
            pltpu.SemaphoreType.DMA,
        ],
        compiler_params=pltpu.CompilerParams(
            use_tc_tiling_on_sc=False, needs_layout_passes=False),
    )
    def sc_scatter(s_hbm, src_hbm, dst_hbm, b2_hbm, out_hbm,
                   s_v, acc_v, cnt_v, src_v, dst_v,
                   red_v, out_v, b2_v, sh_acc, sh_cnt, sem):
        sid = lax.axis_index("s")
        base = sid * ept

        cp_s = pltpu.async_copy(s_hbm, s_v, sem)
        cp_src = pltpu.async_copy(src_hbm.at[pl.ds(base, ept)],
                                  src_v.at[pl.ds(0, ept)], sem)
        cp_dst = pltpu.async_copy(dst_hbm.at[pl.ds(base, ept)],
                                  dst_v.at[pl.ds(0, ept)], sem)
        cp_b2 = pltpu.async_copy(b2_hbm, b2_v, sem)

        zeros = jnp.zeros((_L,), jnp.float32)

        def zero_body(i, carry):
            for u in range(_UNROLL):
                acc_v[pl.ds((i * _UNROLL + u) * _L, _L)] = zeros
                cnt_v[pl.ds((i * _UNROLL + u) * _L, _L)] = zeros
            return carry

        lax.fori_loop(0, n_pad // (_L * _UNROLL), zero_body, 0)

        cp_s.wait()
        cp_src.wait()
        cp_dst.wait()
        cp_b2.wait()

        ones = jnp.ones((_L,), jnp.float32)

        def edge_group(g):
            sidx = src_v[pl.ds(g * _L, _L)]
            didx = dst_v[pl.ds(g * _L, _L)]
            vals = plsc.load_gather(s_v, [sidx])
            plsc.addupdate_scatter(acc_v, [didx], vals)
            plsc.addupdate_scatter(cnt_v, [didx], ones)

        def edge_body(i, carry):
            for u in range(_UNROLL):
                edge_group(i * _UNROLL + u)
            return carry

        lax.fori_loop(0, n_full // _UNROLL, edge_body, 0)
        for g in range(n_full // _UNROLL * _UNROLL, n_full):
            edge_group(g)

        if rem:
            lane = lax.iota(jnp.int32, _L)
            valid = lane < rem
            sidx = jnp.where(valid, src_v[pl.ds(n_full * _L, _L)], 0)
            didx = jnp.where(valid, dst_v[pl.ds(n_full * _L, _L)], n_nodes)
            vals = plsc.load_gather(s_v, [sidx])
            plsc.addupdate_scatter(acc_v, [didx], vals)
            plsc.addupdate_scatter(
                cnt_v, [didx], jnp.where(valid, ones, jnp.zeros((_L,))))

        pltpu.sync_copy(acc_v, sh_acc.at[sid])
        pltpu.sync_copy(cnt_v, sh_cnt.at[sid])
        plsc.subcore_barrier()

        pltpu.sync_copy(sh_acc.at[:, pl.ds(sid * rows, rows)], red_v)

        def red_sum_body(j, carry):
            v = red_v[0, pl.ds(j * _L, _L)]
            for r in range(1, _NS):
                v = v + red_v[r, pl.ds(j * _L, _L)]
            out_v[pl.ds(j * _L, _L)] = v
            return carry

        lax.fori_loop(0, rows // _L, red_sum_body, 0)

        pltpu.sync_copy(sh_cnt.at[:, pl.ds(sid * rows, rows)], red_v)
        b2v = b2_v[...]

        def red_cnt_body(j, carry):
            c = red_v[0, pl.ds(j * _L, _L)]
            for r in range(1, _NS):
                c = c + red_v[r, pl.ds(j * _L, _L)]
            sl = pl.ds(j * _L, _L)
            out_v[sl] = out_v[sl] / jnp.maximum(c, 1.0) + b2v
            return carry

        lax.fori_loop(0, rows // _L, red_cnt_body, 0)

        @pl.when(sid != _NS - 1)
        def _():
            pltpu.sync_copy(out_v, out_hbm.at[pl.ds(sid * rows, rows)])

        @pl.when(sid == _NS - 1)
        def _():
            pltpu.sync_copy(out_v.at[pl.ds(0, tail_rows)],
                            out_hbm.at[pl.ds((_NS - 1) * rows, tail_rows)])

    return sc_scatter


@jax.jit
def kernel(x, edge_index, W1, b1, W2, b2):
    n, _ = x.shape
    e = edge_index.shape[1]

    ept = -(-e // _NS)
    n_pad = _round_up(n + 1, 1024)

    s = _node_scalar(x, W1, b1, W2, block_rows=2000)
    s_flat = s.reshape(n)

    src = edge_index[0].astype(jnp.int32)
    dst = edge_index[1].astype(jnp.int32)
    b2v = jnp.broadcast_to(b2.astype(jnp.float32), (_L,))

    out_vec = _make_sc_scatter(n, n_pad, ept)(s_flat, src, dst, b2v)
    return out_vec[:, None]


# speedup vs baseline: 1.1645x; 1.1052x over previous
import functools

import jax
import jax.numpy as jnp
from jax import lax
from jax.experimental import pallas as pl
from jax.experimental.pallas import tpu as pltpu
from jax.experimental.pallas import tpu_sc as plsc

_NC = 2
_NS = 16
_NW = _NC * _NS
_L = 16


def _round_up(a, b):
    return (a + b - 1) // b * b


def _proj_body(x_ref, w1_ref, b1_ref, w2_ref, s_ref):
    xb = x_ref[...]
    h = jnp.maximum(
        lax.dot_general(xb, w1_ref[...], (((1,), (1,)), ((), ())),
                        preferred_element_type=jnp.float32)
        + b1_ref[...][None, :],
        0.0,
    )
    s_ref[...] = lax.dot_general(h, w2_ref[...], (((1,), (1,)), ((), ())),
                                 preferred_element_type=jnp.float32)


def _node_scalar(x, W1, b1, W2, block_rows):
    n, d_in = x.shape
    d_hid = W1.shape[0]
    grid = n // block_rows
    return pl.pallas_call(
        _proj_body,
        grid=(grid,),
        in_specs=[
            pl.BlockSpec((block_rows, d_in), lambda i: (i, 0)),
            pl.BlockSpec((d_hid, d_in), lambda i: (0, 0)),
            pl.BlockSpec((d_hid,), lambda i: (0,)),
            pl.BlockSpec((1, d_hid), lambda i: (0, 0)),
        ],
        out_specs=pl.BlockSpec((block_rows, 1), lambda i: (i, 0)),
        out_shape=jax.ShapeDtypeStruct((n, 1), jnp.float32),
    )(x, W1, b1, W2)


_UNROLL = 8


def _make_sc_scatter(n_nodes, n_pad, ept):
    mesh = plsc.VectorSubcoreMesh(
        core_axis_name="c", subcore_axis_name="s",
        num_cores=1, num_subcores=_NS)

    ept_pad = _round_up(ept, _L)
    n_full = ept // _L
    rem = ept - n_full * _L
    rows = n_pad // _NS

    tail_rows = n_nodes - (_NS - 1) * rows

    @functools.partial(
        pl.kernel,
        out_type=jax.ShapeDtypeStruct((n_nodes,), jnp.float32),
        mesh=mesh,
        scratch_types=[
            pltpu.VMEM((n_nodes,), jnp.float32),
            pltpu.VMEM((n_pad,), jnp.float32),
            pltpu.VMEM((n_pad,), jnp.float32),
            pltpu.VMEM((ept_pad,), jnp.int32),
            pltpu.VMEM((ept_pad,), jnp.int32),
            pltpu.VMEM((_NS, rows), jnp.float32),
            pltpu.VMEM((rows,), jnp.float32),
            pltpu.VMEM((_L,), jnp.float32),
            pltpu.VMEM_SHARED((_NS, n_pad), jnp.float32),
            pltpu.VMEM_SHARED((_NS, n_pad), jnp.float32),
            pltpu.SemaphoreType.DMA,
        ],
        compiler_params=pltpu.CompilerParams(
            use_tc_tiling_on_sc=False, needs_layout_passes=False),
    )
    def sc_scatter(s_hbm, ei_hbm, b2_hbm, out_hbm,
                   s_v, acc_v, cnt_v, src_v, dst_v,
                   red_v, out_v, b2_v, sh_acc, sh_cnt, sem):
        sid = lax.axis_index("s")
        base = sid * ept

        cp_s = pltpu.async_copy(s_hbm, s_v, sem)
        cp_src = pltpu.async_copy(ei_hbm.at[0, pl.ds(base, ept)],
                                  src_v.at[pl.ds(0, ept)], sem)
        cp_dst = pltpu.async_copy(ei_hbm.at[1, pl.ds(base, ept)],
                                  dst_v.at[pl.ds(0, ept)], sem)
        cp_b2 = pltpu.async_copy(b2_hbm, b2_v, sem)

        zeros = jnp.zeros((_L,), jnp.float32)

        def zero_body(i, carry):
            for u in range(_UNROLL):
                acc_v[pl.ds((i * _UNROLL + u) * _L, _L)] = zeros
                cnt_v[pl.ds((i * _UNROLL + u) * _L, _L)] = zeros
            return carry

        lax.fori_loop(0, n_pad // (_L * _UNROLL), zero_body, 0)

        cp_s.wait()
        cp_src.wait()
        cp_dst.wait()
        cp_b2.wait()

        ones = jnp.ones((_L,), jnp.float32)

        def edge_group(g):
            sidx = src_v[pl.ds(g * _L, _L)]
            didx = dst_v[pl.ds(g * _L, _L)]
            vals = plsc.load_gather(s_v, [sidx])
            plsc.addupdate_scatter(acc_v, [didx], vals)
            plsc.addupdate_scatter(cnt_v, [didx], ones)

        def edge_body(i, carry):
            for u in range(_UNROLL):
                edge_group(i * _UNROLL + u)
            return carry

        lax.fori_loop(0, n_full // _UNROLL, edge_body, 0)
        for g in range(n_full // _UNROLL * _UNROLL, n_full):
            edge_group(g)

        if rem:
            lane = lax.iota(jnp.int32, _L)
            valid = lane < rem
            sidx = jnp.where(valid, src_v[pl.ds(n_full * _L, _L)], 0)
            didx = jnp.where(valid, dst_v[pl.ds(n_full * _L, _L)], n_nodes)
            vals = plsc.load_gather(s_v, [sidx])
            plsc.addupdate_scatter(acc_v, [didx], vals)
            plsc.addupdate_scatter(
                cnt_v, [didx], jnp.where(valid, ones, jnp.zeros((_L,))))

        pltpu.sync_copy(acc_v, sh_acc.at[sid])
        pltpu.sync_copy(cnt_v, sh_cnt.at[sid])
        plsc.subcore_barrier()

        pltpu.sync_copy(sh_acc.at[:, pl.ds(sid * rows, rows)], red_v)

        def red_sum_body(j, carry):
            v = red_v[0, pl.ds(j * _L, _L)]
            for r in range(1, _NS):
                v = v + red_v[r, pl.ds(j * _L, _L)]
            out_v[pl.ds(j * _L, _L)] = v
            return carry

        lax.fori_loop(0, rows // _L, red_sum_body, 0)

        pltpu.sync_copy(sh_cnt.at[:, pl.ds(sid * rows, rows)], red_v)
        b2v = b2_v[...]

        def red_cnt_body(j, carry):
            c = red_v[0, pl.ds(j * _L, _L)]
            for r in range(1, _NS):
                c = c + red_v[r, pl.ds(j * _L, _L)]
            sl = pl.ds(j * _L, _L)
            out_v[sl] = out_v[sl] / jnp.maximum(c, 1.0) + b2v
            return carry

        lax.fori_loop(0, rows // _L, red_cnt_body, 0)

        @pl.when(sid != _NS - 1)
        def _():
            pltpu.sync_copy(out_v, out_hbm.at[pl.ds(sid * rows, rows)])

        @pl.when(sid == _NS - 1)
        def _():
            pltpu.sync_copy(out_v.at[pl.ds(0, tail_rows)],
                            out_hbm.at[pl.ds((_NS - 1) * rows, tail_rows)])

    return sc_scatter


@jax.jit
def kernel(x, edge_index, W1, b1, W2, b2):
    n, _ = x.shape
    e = edge_index.shape[1]

    ept = -(-e // _NS)
    n_pad = _round_up(n + 1, 1024)

    s_flat = _node_scalar(x, W1, b1, W2, block_rows=2000).reshape(n)

    ei = edge_index.astype(jnp.int32)
    b2v = jnp.broadcast_to(b2.astype(jnp.float32), (_L,))

    out_vec = _make_sc_scatter(n, n_pad, ept)(s_flat, ei, b2v)
    return out_vec[:, None]


# speedup vs baseline: 1.2037x; 1.0336x over previous
import functools

import jax
import jax.numpy as jnp
from jax import lax
from jax.experimental import pallas as pl
from jax.experimental.pallas import tpu as pltpu
from jax.experimental.pallas import tpu_sc as plsc

_NC = 2
_NS = 16
_NW = _NC * _NS
_L = 16


def _round_up(a, b):
    return (a + b - 1) // b * b


def _proj_body(x_ref, w1_ref, b1_ref, w2_ref, s_ref):
    xb = x_ref[...]
    h = jnp.maximum(
        lax.dot_general(xb, w1_ref[...], (((1,), (1,)), ((), ())),
                        preferred_element_type=jnp.float32)
        + b1_ref[...][None, :],
        0.0,
    )
    s_ref[...] = lax.dot_general(h, w2_ref[...], (((1,), (1,)), ((), ())),
                                 preferred_element_type=jnp.float32)


def _node_scalar(x, W1, b1, W2, block_rows):
    n, d_in = x.shape
    d_hid = W1.shape[0]
    grid = n // block_rows
    return pl.pallas_call(
        _proj_body,
        grid=(grid,),
        in_specs=[
            pl.BlockSpec((block_rows, d_in), lambda i: (i, 0)),
            pl.BlockSpec((d_hid, d_in), lambda i: (0, 0)),
            pl.BlockSpec((d_hid,), lambda i: (0,)),
            pl.BlockSpec((1, d_hid), lambda i: (0, 0)),
        ],
        out_specs=pl.BlockSpec((block_rows, 1), lambda i: (i, 0)),
        out_shape=jax.ShapeDtypeStruct((n, 1), jnp.float32),
    )(x, W1, b1, W2)


_UNROLL = 8


def _make_sc_scatter(n_nodes, n_pad, ept):
    mesh = plsc.VectorSubcoreMesh(
        core_axis_name="c", subcore_axis_name="s",
        num_cores=1, num_subcores=_NS)

    ept_pad = _round_up(ept, _L)
    n_full = ept // _L
    rem = ept - n_full * _L
    rows = n_pad // _NS

    tail_rows = n_nodes - (_NS - 1) * rows

    @functools.partial(
        pl.kernel,
        out_type=jax.ShapeDtypeStruct((n_nodes,), jnp.float32),
        mesh=mesh,
        scratch_types=[
            pltpu.VMEM((n_nodes,), jnp.float32),
            pltpu.VMEM((n_pad,), jnp.float32),
            pltpu.VMEM((n_pad,), jnp.float32),
            pltpu.VMEM((ept_pad,), jnp.int32),
            pltpu.VMEM((ept_pad,), jnp.int32),
            pltpu.VMEM((_NS, rows), jnp.float32),
            pltpu.VMEM((_NS, rows), jnp.float32),
            pltpu.VMEM((rows,), jnp.float32),
            pltpu.VMEM((_L,), jnp.float32),
            pltpu.VMEM_SHARED((_NS, n_pad), jnp.float32),
            pltpu.VMEM_SHARED((_NS, n_pad), jnp.float32),
            pltpu.SemaphoreType.DMA,
        ],
        compiler_params=pltpu.CompilerParams(
            use_tc_tiling_on_sc=False, needs_layout_passes=False),
    )
    def sc_scatter(s_hbm, ei_hbm, b2_hbm, out_hbm,
                   s_v, acc_v, cnt_v, src_v, dst_v,
                   red_v, redc_v, out_v, b2_v, sh_acc, sh_cnt, sem):
        sid = lax.axis_index("s")
        base = sid * ept

        cp_s = pltpu.async_copy(s_hbm, s_v, sem)
        cp_src = pltpu.async_copy(ei_hbm.at[0, pl.ds(base, ept)],
                                  src_v.at[pl.ds(0, ept)], sem)
        cp_dst = pltpu.async_copy(ei_hbm.at[1, pl.ds(base, ept)],
                                  dst_v.at[pl.ds(0, ept)], sem)
        cp_b2 = pltpu.async_copy(b2_hbm, b2_v.at[pl.ds(0, 1)], sem)

        zeros = jnp.zeros((_L,), jnp.float32)

        def zero_body(i, carry):
            for u in range(_UNROLL):
                acc_v[pl.ds((i * _UNROLL + u) * _L, _L)] = zeros
                cnt_v[pl.ds((i * _UNROLL + u) * _L, _L)] = zeros
            return carry

        lax.fori_loop(0, n_pad // (_L * _UNROLL), zero_body, 0)

        cp_s.wait()
        cp_src.wait()
        cp_dst.wait()
        cp_b2.wait()

        ones = jnp.ones((_L,), jnp.float32)

        def edge_group(g):
            sidx = src_v[pl.ds(g * _L, _L)]
            didx = dst_v[pl.ds(g * _L, _L)]
            vals = plsc.load_gather(s_v, [sidx])
            plsc.addupdate_scatter(acc_v, [didx], vals)
            plsc.addupdate_scatter(cnt_v, [didx], ones)

        def edge_body(i, carry):
            for u in range(_UNROLL):
                edge_group(i * _UNROLL + u)
            return carry

        lax.fori_loop(0, n_full // _UNROLL, edge_body, 0)
        for g in range(n_full // _UNROLL * _UNROLL, n_full):
            edge_group(g)

        if rem:
            lane = lax.iota(jnp.int32, _L)
            valid = lane < rem
            sidx = jnp.where(valid, src_v[pl.ds(n_full * _L, _L)], 0)
            didx = jnp.where(valid, dst_v[pl.ds(n_full * _L, _L)], n_nodes)
            vals = plsc.load_gather(s_v, [sidx])
            plsc.addupdate_scatter(acc_v, [didx], vals)
            plsc.addupdate_scatter(
                cnt_v, [didx], jnp.where(valid, ones, jnp.zeros((_L,))))

        cp_pa = pltpu.async_copy(acc_v, sh_acc.at[sid], sem)
        cp_pc = pltpu.async_copy(cnt_v, sh_cnt.at[sid], sem)
        cp_pa.wait()
        cp_pc.wait()
        plsc.subcore_barrier()

        cp_ra = pltpu.async_copy(sh_acc.at[:, pl.ds(sid * rows, rows)],
                                 red_v, sem)
        cp_rc = pltpu.async_copy(sh_cnt.at[:, pl.ds(sid * rows, rows)],
                                 redc_v, sem)
        cp_ra.wait()

        def red_sum_body(j, carry):
            v = red_v[0, pl.ds(j * _L, _L)]
            for r in range(1, _NS):
                v = v + red_v[r, pl.ds(j * _L, _L)]
            out_v[pl.ds(j * _L, _L)] = v
            return carry

        lax.fori_loop(0, rows // _L, red_sum_body, 0)

        cp_rc.wait()
        b2v = jnp.full((_L,), b2_v[...][0], jnp.float32)

        def red_cnt_body(j, carry):
            c = redc_v[0, pl.ds(j * _L, _L)]
            for r in range(1, _NS):
                c = c + redc_v[r, pl.ds(j * _L, _L)]
            sl = pl.ds(j * _L, _L)
            out_v[sl] = out_v[sl] / jnp.maximum(c, 1.0) + b2v
            return carry

        lax.fori_loop(0, rows // _L, red_cnt_body, 0)

        @pl.when(sid != _NS - 1)
        def _():
            pltpu.sync_copy(out_v, out_hbm.at[pl.ds(sid * rows, rows)])

        @pl.when(sid == _NS - 1)
        def _():
            pltpu.sync_copy(out_v.at[pl.ds(0, tail_rows)],
                            out_hbm.at[pl.ds((_NS - 1) * rows, tail_rows)])

    return sc_scatter


@jax.jit
def kernel(x, edge_index, W1, b1, W2, b2):
    n, _ = x.shape
    e = edge_index.shape[1]

    ept = -(-e // _NS)
    n_pad = _round_up(n + 1, 1024)

    s_flat = _node_scalar(x, W1, b1, W2, block_rows=2000).reshape(n)

    ei = edge_index.astype(jnp.int32)

    out_vec = _make_sc_scatter(n, n_pad, ept)(s_flat, ei, b2.astype(jnp.float32))
    return out_vec[:, None]
